# Initial kernel scaffold; baseline (speedup 1.0000x reference)
import jax
import jax.numpy as jnp
from jax.experimental import pallas as pl


def kernel(anchor, loc, score, img_size):
    raise NotImplementedError("write your pallas kernel here")


# speedup vs baseline: 17.8894x; 17.8894x over previous
import functools

import jax
import jax.numpy as jnp
from jax import lax
from jax.experimental import pallas as pl
from jax.experimental.pallas import tpu as pltpu
from jax.experimental.pallas import tpu_sc as plsc

_N = 20000
_ROWS = 160
_COLS = 128
_NPAD = _ROWS * _COLS
_N_PRE = 6000
_N_POST = 300
_OUT_ROWS = 304
_NMS_THRESH = 0.7
_FEAT_STRIDE = 16.0
_NEG = -1e30
_INT_MIN = -2147483648

_NSC = 16
_NT_SCAT = 10
_TROWS = _ROWS // _NT_SCAT
_CAP = 6144
_CROWS = 48
_PER_OUT = _CAP // _NSC
_CPAD = _CAP + 8


def _rsum(x):
    return jnp.sum(jnp.sum(x, axis=0, keepdims=True), axis=1, keepdims=True)


def _rmax(x):
    return jnp.max(jnp.max(x, axis=0, keepdims=True), axis=1, keepdims=True)


def _rmin(x):
    return jnp.min(jnp.min(x, axis=0, keepdims=True), axis=1, keepdims=True)


def _bc(x, shape=(_ROWS, _COLS)):
    return jnp.broadcast_to(x, shape)


def _select_body(anc_ref, loc_ref, sc_ref, img_ref,
                 pos_o, idx_o, y1_o, x1_o, y2_o, x2_o, ar_o, cs_o, fill_o):
    hI = img_ref[0].astype(jnp.float32)
    wI = img_ref[1].astype(jnp.float32)

    ay1 = anc_ref[0]
    ax1 = anc_ref[1]
    ay2 = anc_ref[2]
    ax2 = anc_ref[3]
    dy = loc_ref[0]
    dx = loc_ref[1]
    dh = loc_ref[2]
    dw = loc_ref[3]

    src_h = ay2 - ay1
    src_w = ax2 - ax1
    src_cy = ay1 + 0.5 * src_h
    src_cx = ax1 + 0.5 * src_w
    cy = dy * src_h + src_cy
    cx = dx * src_w + src_cx
    h = jnp.exp(dh) * src_h
    w = jnp.exp(dw) * src_w
    y1 = jnp.clip(cy - 0.5 * h, 0.0, hI)
    x1 = jnp.clip(cx - 0.5 * w, 0.0, wI)
    y2 = jnp.clip(cy + 0.5 * h, 0.0, hI)
    x2 = jnp.clip(cx + 0.5 * w, 0.0, wI)
    hh = y2 - y1
    ww = x2 - x1
    valid = (hh >= _FEAT_STRIDE) & (ww >= _FEAT_STRIDE)
    score_f = jnp.where(valid, sc_ref[...], -1e9)

    idx = (lax.broadcasted_iota(jnp.int32, (_ROWS, _COLS), 0) * _COLS
           + lax.broadcasted_iota(jnp.int32, (_ROWS, _COLS), 1))
    in_range = idx < _N
    bits = lax.bitcast_convert_type(score_f, jnp.int32)
    key = jnp.where(bits >= 0, bits, bits ^ 0x7FFFFFFF)
    key = jnp.where(in_range, key, _INT_MIN)

    cnt0 = _rsum((key >= 0).astype(jnp.int32))
    v = jnp.where(cnt0 >= _N_PRE, 0, _INT_MIN).astype(jnp.int32)
    for b in range(30, -1, -1):
        cand = v | (1 << b)
        cnt = _rsum((key >= _bc(cand)).astype(jnp.int32))
        v = jnp.where(cnt >= _N_PRE, cand, v)
    tau = v

    tie = key == _bc(tau)
    cnt_ge = _rsum((key >= _bc(tau)).astype(jnp.int32))
    cnt_eq = _rsum(tie.astype(jnp.int32))
    needed = _N_PRE - (cnt_ge - cnt_eq)

    u = jnp.zeros((1, 1), jnp.int32)
    for b in range(14, -1, -1):
        cand = u | (1 << b)
        c = _rsum((tie & (idx < _bc(cand))).astype(jnp.int32))
        u = jnp.where(c < needed, cand, u)
    ucut = u

    member = (key > _bc(tau)) | (tie & (idx <= _bc(ucut)))
    alive = member & (score_f > -1e8)
    cs = jnp.where(alive, score_f, _NEG)

    areas = jnp.maximum(x2 - x1, 0.0) * jnp.maximum(y2 - y1, 0.0)

    a = alive.astype(jnp.int32)
    rp = a
    for d in (1, 2, 4, 8, 16, 32, 64):
        rp = rp + jnp.concatenate(
            [jnp.zeros((_ROWS, d), jnp.int32), rp[:, :_COLS - d]], axis=1)
    rt = rp[:, _COLS - 1:_COLS]
    ip = rt
    for d in (1, 2, 4, 8, 16, 32, 64, 128):
        ip = ip + jnp.concatenate(
            [jnp.zeros((d, 1), jnp.int32), ip[:_ROWS - d, :]], axis=0)
    pos = (ip - rt) + rp - a
    pos_o[...] = jnp.where(alive, pos, _CAP)
    idx_o[...] = idx

    maxkey = _rmax(key)
    j0 = _rmin(jnp.where(key == _bc(maxkey), idx, 1 << 30))
    oh0 = idx == _bc(j0)
    f_y1 = _rsum(jnp.where(oh0, y1, 0.0))
    f_x1 = _rsum(jnp.where(oh0, x1, 0.0))
    f_y2 = _rsum(jnp.where(oh0, y2, 0.0))
    f_x2 = _rsum(jnp.where(oh0, x2, 0.0))
    lane = lax.broadcasted_iota(jnp.int32, (1, _COLS), 1)
    frow = jnp.where(
        lane == 0, _bc(f_y1, (1, _COLS)),
        jnp.where(lane == 1, _bc(f_x1, (1, _COLS)),
                  jnp.where(lane == 2, _bc(f_y2, (1, _COLS)),
                            jnp.where(lane == 3, _bc(f_x2, (1, _COLS)),
                                      jnp.zeros((1, _COLS), jnp.float32)))))

    y1_o[...] = y1
    x1_o[...] = x1
    y2_o[...] = y2
    x2_o[...] = x2
    ar_o[...] = areas
    cs_o[...] = cs
    fill_o[...] = jnp.broadcast_to(frow, (8, _COLS))


def _compact_body(pos_h, idx_h, cs_h, y1_h, x1_h, y2_h, x2_h, ar_h,
                  cidx_h, ccs_h, cy1_h, cx1_h, cy2_h, cx2_h, car_h,
                  posb, pib, pf0, pf1, pf2, pf3, pf4, pf5,
                  inii, inif, sem):
    cid = lax.axis_index("c")
    sid = lax.axis_index("s")

    @pl.when(cid == 0)
    def _work():
        ob = sid * _PER_OUT

        for k in range(_PER_OUT // 16):
            inii[pl.ds(k * 16, 16)] = jnp.full((16,), _NPAD - 1, jnp.int32)
            inif[pl.ds(k * 16, 16)] = jnp.full((16,), _NEG, jnp.float32)
        fbufs = [pf0, pf1, pf2, pf3, pf4, pf5]
        fsrcs = [cs_h, y1_h, x1_h, y2_h, x2_h, ar_h]
        hs = [pltpu.async_copy(inii, cidx_h.at[pl.ds(ob, _PER_OUT)], sem),
              pltpu.async_copy(inif, ccs_h.at[pl.ds(ob, _PER_OUT)], sem)]
        for h in hs:
            h.wait()

        @pl.when(sid < _NT_SCAT)
        def _stage():
            rb = sid * _TROWS
            hs2 = [pltpu.async_copy(pos_h.at[pl.ds(rb, _TROWS)], posb, sem),
                   pltpu.async_copy(idx_h.at[pl.ds(rb, _TROWS)], pib, sem)]
            for src, buf in zip(fsrcs, fbufs):
                hs2.append(pltpu.async_copy(src.at[pl.ds(rb, _TROWS)], buf,
                                            sem))
            for h in hs2:
                h.wait()

        plsc.subcore_barrier()

        @pl.when(sid < _NT_SCAT)
        def _scatter():
            fdsts = [ccs_h, cy1_h, cx1_h, cy2_h, cx2_h, car_h]
            hs3 = []
            for j in range(_TROWS):
                hs3.append(pltpu.async_copy(pib.at[j], cidx_h.at[posb.at[j]],
                                            sem))
                for buf, dst in zip(fbufs, fdsts):
                    hs3.append(pltpu.async_copy(buf.at[j],
                                                dst.at[posb.at[j]], sem))
            for h in hs3:
                h.wait()


_compact = functools.partial(
    pl.kernel,
    mesh=plsc.VectorSubcoreMesh(core_axis_name="c", subcore_axis_name="s"),
    out_type=[jax.ShapeDtypeStruct((_CPAD,), jnp.int32)]
    + [jax.ShapeDtypeStruct((_CPAD,), jnp.float32) for _ in range(6)],
    scratch_types=[
        pltpu.VMEM((_TROWS, _COLS), jnp.int32),
        pltpu.VMEM((_TROWS, _COLS), jnp.int32),
    ]
    + [pltpu.VMEM((_TROWS, _COLS), jnp.float32) for _ in range(6)]
    + [
        pltpu.VMEM((_PER_OUT,), jnp.int32),
        pltpu.VMEM((_PER_OUT,), jnp.float32),
        pltpu.SemaphoreType.DMA,
    ],
)(_compact_body)


def _nms_body(cidx_ref, y1_ref, x1_ref, y2_ref, x2_ref, ar_ref, cs_ref,
              fill_ref, out_ref, cs_s):
    cs_s[...] = cs_ref[...]
    cidx = cidx_ref[...]
    frow = fill_ref[0:1, :]
    lane = lax.broadcasted_iota(jnp.int32, (1, _COLS), 1)

    def _shape(x):
        return jnp.broadcast_to(x, (_CROWS, _COLS))

    def _cmax(x):
        return jnp.max(jnp.max(x, axis=0, keepdims=True), axis=1,
                       keepdims=True)

    def _cmin(x):
        return jnp.min(jnp.min(x, axis=0, keepdims=True), axis=1,
                       keepdims=True)

    def _csum(x):
        return jnp.sum(jnp.sum(x, axis=0, keepdims=True), axis=1,
                       keepdims=True)

    def _nms_step(k, carry):
        s = cs_s[...]
        m = _cmax(s)
        found = m > -1e20
        curi = _cmin(jnp.where(s == _shape(m), cidx, 1 << 30))
        oh = cidx == _shape(curi)
        cy1 = _csum(jnp.where(oh, y1_ref[...], 0.0))
        cx1 = _csum(jnp.where(oh, x1_ref[...], 0.0))
        cy2 = _csum(jnp.where(oh, y2_ref[...], 0.0))
        cx2 = _csum(jnp.where(oh, x2_ref[...], 0.0))
        car = _csum(jnp.where(oh, ar_ref[...], 0.0))
        row = jnp.where(
            lane == 0, jnp.broadcast_to(cy1, (1, _COLS)),
            jnp.where(lane == 1, jnp.broadcast_to(cx1, (1, _COLS)),
                      jnp.where(lane == 2, jnp.broadcast_to(cy2, (1, _COLS)),
                                jnp.broadcast_to(cx2, (1, _COLS)))))
        out_ref[pl.ds(k, 1), :] = jnp.where(
            jnp.broadcast_to(found, (1, _COLS)), row, frow)
        xx1 = jnp.maximum(_shape(cx1), x1_ref[...])
        yy1 = jnp.maximum(_shape(cy1), y1_ref[...])
        xx2 = jnp.minimum(_shape(cx2), x2_ref[...])
        yy2 = jnp.minimum(_shape(cy2), y2_ref[...])
        inter = jnp.maximum(xx2 - xx1, 0.0) * jnp.maximum(yy2 - yy1, 0.0)
        iou = inter / (_shape(car) + ar_ref[...] - inter + 1e-9)
        supp = (iou > _NMS_THRESH) | oh
        cs_s[...] = jnp.where(_shape(found) & supp, _NEG, s)
        return carry

    lax.fori_loop(0, _N_POST, _nms_step, 0)


@jax.jit
def kernel(anchor, loc, score, img_size):
    anc = jnp.pad(anchor, ((0, _NPAD - _N), (0, 0))).T.reshape(4, _ROWS, _COLS)
    locp = jnp.pad(loc, ((0, _NPAD - _N), (0, 0))).T.reshape(4, _ROWS, _COLS)
    scp = jnp.pad(score, (0, _NPAD - _N)).reshape(_ROWS, _COLS)

    pos, idxg, y1, x1, y2, x2, ar, cs, fill = pl.pallas_call(
        _select_body,
        out_shape=[jax.ShapeDtypeStruct((_ROWS, _COLS), jnp.int32)
                   for _ in range(2)]
        + [jax.ShapeDtypeStruct((_ROWS, _COLS), jnp.float32)
           for _ in range(6)]
        + [jax.ShapeDtypeStruct((8, _COLS), jnp.float32)],
        in_specs=[
            pl.BlockSpec(memory_space=pltpu.VMEM),
            pl.BlockSpec(memory_space=pltpu.VMEM),
            pl.BlockSpec(memory_space=pltpu.VMEM),
            pl.BlockSpec(memory_space=pltpu.SMEM),
        ],
        out_specs=[pl.BlockSpec(memory_space=pltpu.VMEM)
                   for _ in range(9)],
    )(anc, locp, scp, img_size)

    cidx, ccs, cy1, cx1, cy2, cx2, car = _compact(
        pos, idxg, cs, y1, x1, y2, x2, ar)

    out = pl.pallas_call(
        _nms_body,
        out_shape=jax.ShapeDtypeStruct((_OUT_ROWS, _COLS), jnp.float32),
        in_specs=[pl.BlockSpec(memory_space=pltpu.VMEM)
                  for _ in range(8)],
        out_specs=pl.BlockSpec(memory_space=pltpu.VMEM),
        scratch_shapes=[pltpu.VMEM((_CROWS, _COLS), jnp.float32)],
    )(cidx[:_CAP].reshape(_CROWS, _COLS),
      cy1[:_CAP].reshape(_CROWS, _COLS), cx1[:_CAP].reshape(_CROWS, _COLS),
      cy2[:_CAP].reshape(_CROWS, _COLS), cx2[:_CAP].reshape(_CROWS, _COLS),
      car[:_CAP].reshape(_CROWS, _COLS), ccs[:_CAP].reshape(_CROWS, _COLS),
      fill)
    return out[:_N_POST, :4]
